# baseline (speedup 1.0000x reference)
import functools

import jax
import jax.numpy as jnp
from jax.experimental import pallas as pl
from jax.experimental.pallas import tpu as pltpu

B = 8; S = 8; NT = 4; T = 128; D = 64; K = 256; V = 20000
VT = 2048


def _body(subj_ref, task_ref, locT, fmu, fsig, smu, ssig, tmu, tsig,
          epsF, epsP, epsS, wc0, wc1, wc2, ww, wtop, wbot, y_ref):
    z_f = fmu[0] + fsig[0] * epsF[0]
    z_p = smu[0] + ssig[0] * epsP[0]
    z_s = tmu[0] + tsig[0] * epsS[0]

    i0 = jax.lax.broadcasted_iota(jnp.int32, (D, D), 0)
    i1 = jax.lax.broadcasted_iota(jnp.int32, (D, D), 1)
    eye = jnp.where(i0 == i1, 1.0, 0.0).astype(jnp.float32)
    z_col = jnp.sum(jnp.broadcast_to(z_f, (D, D)) * eye, axis=1,
                    keepdims=True)

    c0 = jnp.dot(wc0[...], z_col, preferred_element_type=jnp.float32)
    c1 = jnp.dot(wc1[...], z_col, preferred_element_type=jnp.float32)
    c2 = jnp.dot(wc2[...], z_col, preferred_element_type=jnp.float32)
    logw = jnp.dot(ww[...], z_col, preferred_element_type=jnp.float32) + 2.0
    invw = jnp.exp(-logw)

    wrow = jnp.dot(z_p, wtop[...], preferred_element_type=jnp.float32)
    wts = jnp.dot(z_s, wbot[...], preferred_element_type=jnp.float32) + wrow

    x0 = locT[0:1, :]
    x1 = locT[1:2, :]
    x2 = locT[2:3, :]
    d0 = c0 - x0
    d1 = c1 - x1
    d2 = c2 - x2
    dist2 = d0 * d0 + d1 * d1 + d2 * d2
    f = jnp.exp(-(dist2 * invw))

    y_ref[0] = jnp.dot(wts, f, preferred_element_type=jnp.float32)


@functools.partial(jax.jit, static_argnums=())
def kernel(locations, block_subjects, block_tasks, factors_mu, factors_sigma,
           subject_mu, subject_sigma, task_mu, task_sigma, eps_F, eps_P,
           eps_S, W_c, W_w, W_wt):
    locT = locations.T
    wc = W_c.reshape(D, K, 3)
    wc0T = wc[:, :, 0].T
    wc1T = wc[:, :, 1].T
    wc2T = wc[:, :, 2].T
    wwT = W_w.T
    wtop = W_wt[:D]
    wbot = W_wt[D:]
    fmu3 = factors_mu[:, None, :]
    fsig3 = factors_sigma[:, None, :]
    smu3 = subject_mu[:, None, :]
    ssig3 = subject_sigma[:, None, :]
    epsF3 = eps_F[:, None, :]
    epsP3 = eps_P[:, None, :]

    nv = pl.cdiv(V, VT)
    grid_spec = pltpu.PrefetchScalarGridSpec(
        num_scalar_prefetch=2,
        grid=(B, nv),
        in_specs=[
            pl.BlockSpec((3, VT), lambda b, v, s, t: (0, v)),
            pl.BlockSpec((1, 1, D), lambda b, v, s, t: (s[b], 0, 0)),
            pl.BlockSpec((1, 1, D), lambda b, v, s, t: (s[b], 0, 0)),
            pl.BlockSpec((1, 1, D), lambda b, v, s, t: (s[b], 0, 0)),
            pl.BlockSpec((1, 1, D), lambda b, v, s, t: (s[b], 0, 0)),
            pl.BlockSpec((1, T, D), lambda b, v, s, t: (t[b], 0, 0)),
            pl.BlockSpec((1, T, D), lambda b, v, s, t: (t[b], 0, 0)),
            pl.BlockSpec((1, 1, D), lambda b, v, s, t: (b, 0, 0)),
            pl.BlockSpec((1, 1, D), lambda b, v, s, t: (b, 0, 0)),
            pl.BlockSpec((1, T, D), lambda b, v, s, t: (b, 0, 0)),
            pl.BlockSpec((K, D), lambda b, v, s, t: (0, 0)),
            pl.BlockSpec((K, D), lambda b, v, s, t: (0, 0)),
            pl.BlockSpec((K, D), lambda b, v, s, t: (0, 0)),
            pl.BlockSpec((K, D), lambda b, v, s, t: (0, 0)),
            pl.BlockSpec((D, K), lambda b, v, s, t: (0, 0)),
            pl.BlockSpec((D, K), lambda b, v, s, t: (0, 0)),
        ],
        out_specs=pl.BlockSpec((1, T, VT), lambda b, v, s, t: (b, 0, v)),
    )
    y = pl.pallas_call(
        _body,
        grid_spec=grid_spec,
        out_shape=jax.ShapeDtypeStruct((B, T, V), jnp.float32),
        compiler_params=pltpu.CompilerParams(
            dimension_semantics=("parallel", "parallel"),
        ),
    )(block_subjects, block_tasks, locT, fmu3, fsig3, smu3, ssig3,
      task_mu, task_sigma, epsF3, epsP3, eps_S, wc0T, wc1T, wc2T, wwT,
      wtop, wbot)
    return y


# speedup vs baseline: 1.1797x; 1.1797x over previous
import jax
import jax.numpy as jnp
from jax.experimental import pallas as pl
from jax.experimental.pallas import tpu as pltpu

B = 8; S = 8; NT = 4; T = 128; D = 64; K = 256; V = 20000
VT = 2048


def _body(subj_ref, task_ref, xp, fmu, fsig, smu, ssig, tmu, tsig,
          epsF, epsP, epsS, wc0, wc1, wc2, ww, wtop, wbot, y_ref):
    z_f = fmu[0] + fsig[0] * epsF[0]
    z_p = smu[0] + ssig[0] * epsP[0]
    z_s = tmu[0] + tsig[0] * epsS[0]

    i0 = jax.lax.broadcasted_iota(jnp.int32, (D, D), 0)
    i1 = jax.lax.broadcasted_iota(jnp.int32, (D, D), 1)
    eye = jnp.where(i0 == i1, 1.0, 0.0).astype(jnp.float32)
    z_col = jnp.sum(jnp.broadcast_to(z_f, (D, D)) * eye, axis=1,
                    keepdims=True)

    c0 = jnp.dot(wc0[...], z_col, preferred_element_type=jnp.float32)
    c1 = jnp.dot(wc1[...], z_col, preferred_element_type=jnp.float32)
    c2 = jnp.dot(wc2[...], z_col, preferred_element_type=jnp.float32)
    logw = jnp.dot(ww[...], z_col, preferred_element_type=jnp.float32) + 2.0
    invw = jnp.exp(-logw)

    n2iw = -2.0 * invw
    cenw = invw * (c0 * c0 + c1 * c1 + c2 * c2)
    zcol8 = jnp.zeros((K, 3), dtype=jnp.float32)
    caug = jnp.concatenate(
        [n2iw * c0, n2iw * c1, n2iw * c2, invw, cenw, zcol8], axis=1)

    wrow = jnp.dot(z_p, wtop[...], preferred_element_type=jnp.float32)
    wts = jnp.dot(z_s, wbot[...], preferred_element_type=jnp.float32) + wrow

    x = xp[...]
    ls = jnp.sum(x * x, axis=0, keepdims=True)
    ones = jnp.ones((1, VT), dtype=jnp.float32)
    zrow3 = jnp.zeros((3, VT), dtype=jnp.float32)
    xaug = jnp.concatenate([x[0:3], ls, ones, zrow3], axis=0)

    arg = jnp.dot(caug, xaug, preferred_element_type=jnp.float32)
    f = jnp.exp(-arg)
    y_ref[0] = jnp.dot(wts, f, preferred_element_type=jnp.float32)


def kernel(locations, block_subjects, block_tasks, factors_mu, factors_sigma,
           subject_mu, subject_sigma, task_mu, task_sigma, eps_F, eps_P,
           eps_S, W_c, W_w, W_wt):
    locT = jnp.pad(locations.T, ((0, 5), (0, 0)))
    wc = W_c.reshape(D, K, 3)
    wc0T = wc[:, :, 0].T
    wc1T = wc[:, :, 1].T
    wc2T = wc[:, :, 2].T
    wwT = W_w.T
    wtop = W_wt[:D]
    wbot = W_wt[D:]
    fmu3 = factors_mu[:, None, :]
    fsig3 = factors_sigma[:, None, :]
    smu3 = subject_mu[:, None, :]
    ssig3 = subject_sigma[:, None, :]
    epsF3 = eps_F[:, None, :]
    epsP3 = eps_P[:, None, :]

    nv = pl.cdiv(V, VT)
    grid_spec = pltpu.PrefetchScalarGridSpec(
        num_scalar_prefetch=2,
        grid=(B, nv),
        in_specs=[
            pl.BlockSpec((8, VT), lambda b, v, s, t: (0, v)),
            pl.BlockSpec((1, 1, D), lambda b, v, s, t: (s[b], 0, 0)),
            pl.BlockSpec((1, 1, D), lambda b, v, s, t: (s[b], 0, 0)),
            pl.BlockSpec((1, 1, D), lambda b, v, s, t: (s[b], 0, 0)),
            pl.BlockSpec((1, 1, D), lambda b, v, s, t: (s[b], 0, 0)),
            pl.BlockSpec((1, T, D), lambda b, v, s, t: (t[b], 0, 0)),
            pl.BlockSpec((1, T, D), lambda b, v, s, t: (t[b], 0, 0)),
            pl.BlockSpec((1, 1, D), lambda b, v, s, t: (b, 0, 0)),
            pl.BlockSpec((1, 1, D), lambda b, v, s, t: (b, 0, 0)),
            pl.BlockSpec((1, T, D), lambda b, v, s, t: (b, 0, 0)),
            pl.BlockSpec((K, D), lambda b, v, s, t: (0, 0)),
            pl.BlockSpec((K, D), lambda b, v, s, t: (0, 0)),
            pl.BlockSpec((K, D), lambda b, v, s, t: (0, 0)),
            pl.BlockSpec((K, D), lambda b, v, s, t: (0, 0)),
            pl.BlockSpec((D, K), lambda b, v, s, t: (0, 0)),
            pl.BlockSpec((D, K), lambda b, v, s, t: (0, 0)),
        ],
        out_specs=pl.BlockSpec((1, T, VT), lambda b, v, s, t: (b, 0, v)),
    )
    y = pl.pallas_call(
        _body,
        grid_spec=grid_spec,
        out_shape=jax.ShapeDtypeStruct((B, T, V), jnp.float32),
        compiler_params=pltpu.CompilerParams(
            dimension_semantics=("parallel", "parallel"),
        ),
    )(block_subjects, block_tasks, locT, fmu3, fsig3, smu3, ssig3,
      task_mu, task_sigma, epsF3, epsP3, eps_S, wc0T, wc1T, wc2T, wwT,
      wtop, wbot)
    return y


# speedup vs baseline: 1.3240x; 1.1223x over previous
import jax
import jax.numpy as jnp
from jax.experimental import pallas as pl
from jax.experimental.pallas import tpu as pltpu

B = 8; S = 8; NT = 4; T = 128; D = 64; K = 256; V = 20000
VT = 2048
LOG2E = 1.4426950408889634


def _body(subj_ref, task_ref, xp, fmu, fsig, smu, ssig, tmu, tsig,
          epsF, epsP, epsS, wc0, wc1, wc2, ww, wtop, wbot, y_ref,
          caug_s, wts_s):
    v = pl.program_id(1)

    @pl.when(v == 0)
    def _precompute():
        z_f = fmu[0] + fsig[0] * epsF[0]
        z_p = smu[0] + ssig[0] * epsP[0]
        z_s = tmu[0] + tsig[0] * epsS[0]

        i0 = jax.lax.broadcasted_iota(jnp.int32, (D, D), 0)
        i1 = jax.lax.broadcasted_iota(jnp.int32, (D, D), 1)
        eye = jnp.where(i0 == i1, 1.0, 0.0).astype(jnp.float32)
        z_col = jnp.sum(jnp.broadcast_to(z_f, (D, D)) * eye, axis=1,
                        keepdims=True)

        f32 = jnp.float32
        c0 = jnp.dot(wc0[...], z_col, preferred_element_type=f32)
        c1 = jnp.dot(wc1[...], z_col, preferred_element_type=f32)
        c2 = jnp.dot(wc2[...], z_col, preferred_element_type=f32)
        logw = jnp.dot(ww[...], z_col, preferred_element_type=f32) + 2.0
        invw = jnp.exp(-logw)

        iwl = LOG2E * invw
        iw2 = 2.0 * iwl
        cenw = -iwl * (c0 * c0 + c1 * c1 + c2 * c2)
        zcol3 = jnp.zeros((K, 3), dtype=f32)
        caug_s[...] = jnp.concatenate(
            [iw2 * c0, iw2 * c1, iw2 * c2, -iwl, cenw, zcol3], axis=1)

        wrow = jnp.dot(z_p, wtop[...], preferred_element_type=f32)
        wts_s[...] = jnp.dot(z_s, wbot[...], preferred_element_type=f32) + wrow

    x = xp[...]
    ls = jnp.sum(x * x, axis=0, keepdims=True)
    ones = jnp.ones((1, VT), dtype=jnp.float32)
    zrow3 = jnp.zeros((3, VT), dtype=jnp.float32)
    xaug = jnp.concatenate([x[0:3], ls, ones, zrow3], axis=0)

    arg = jnp.dot(caug_s[...], xaug, preferred_element_type=jnp.float32)
    f = jnp.exp2(arg)
    y_ref[0] = jnp.dot(wts_s[...], f, preferred_element_type=jnp.float32)


def kernel(locations, block_subjects, block_tasks, factors_mu, factors_sigma,
           subject_mu, subject_sigma, task_mu, task_sigma, eps_F, eps_P,
           eps_S, W_c, W_w, W_wt):
    locT = jnp.pad(locations.T, ((0, 5), (0, 0)))
    wc = W_c.reshape(D, K, 3)
    wc0T = wc[:, :, 0].T
    wc1T = wc[:, :, 1].T
    wc2T = wc[:, :, 2].T
    wwT = W_w.T
    wtop = W_wt[:D]
    wbot = W_wt[D:]
    fmu3 = factors_mu[:, None, :]
    fsig3 = factors_sigma[:, None, :]
    smu3 = subject_mu[:, None, :]
    ssig3 = subject_sigma[:, None, :]
    epsF3 = eps_F[:, None, :]
    epsP3 = eps_P[:, None, :]

    nv = pl.cdiv(V, VT)
    grid_spec = pltpu.PrefetchScalarGridSpec(
        num_scalar_prefetch=2,
        grid=(B, nv),
        in_specs=[
            pl.BlockSpec((8, VT), lambda b, v, s, t: (0, v)),
            pl.BlockSpec((1, 1, D), lambda b, v, s, t: (s[b], 0, 0)),
            pl.BlockSpec((1, 1, D), lambda b, v, s, t: (s[b], 0, 0)),
            pl.BlockSpec((1, 1, D), lambda b, v, s, t: (s[b], 0, 0)),
            pl.BlockSpec((1, 1, D), lambda b, v, s, t: (s[b], 0, 0)),
            pl.BlockSpec((1, T, D), lambda b, v, s, t: (t[b], 0, 0)),
            pl.BlockSpec((1, T, D), lambda b, v, s, t: (t[b], 0, 0)),
            pl.BlockSpec((1, 1, D), lambda b, v, s, t: (b, 0, 0)),
            pl.BlockSpec((1, 1, D), lambda b, v, s, t: (b, 0, 0)),
            pl.BlockSpec((1, T, D), lambda b, v, s, t: (b, 0, 0)),
            pl.BlockSpec((K, D), lambda b, v, s, t: (0, 0)),
            pl.BlockSpec((K, D), lambda b, v, s, t: (0, 0)),
            pl.BlockSpec((K, D), lambda b, v, s, t: (0, 0)),
            pl.BlockSpec((K, D), lambda b, v, s, t: (0, 0)),
            pl.BlockSpec((D, K), lambda b, v, s, t: (0, 0)),
            pl.BlockSpec((D, K), lambda b, v, s, t: (0, 0)),
        ],
        out_specs=pl.BlockSpec((1, T, VT), lambda b, v, s, t: (b, 0, v)),
        scratch_shapes=[
            pltpu.VMEM((K, 8), jnp.float32),
            pltpu.VMEM((T, K), jnp.float32),
        ],
    )
    y = pl.pallas_call(
        _body,
        grid_spec=grid_spec,
        out_shape=jax.ShapeDtypeStruct((B, T, V), jnp.float32),
        compiler_params=pltpu.CompilerParams(
            dimension_semantics=("parallel", "arbitrary"),
        ),
    )(block_subjects, block_tasks, locT, fmu3, fsig3, smu3, ssig3,
      task_mu, task_sigma, epsF3, epsP3, eps_S, wc0T, wc1T, wc2T, wwT,
      wtop, wbot)
    return y


# speedup vs baseline: 1.5377x; 1.1614x over previous
import jax
import jax.numpy as jnp
from jax.experimental import pallas as pl
from jax.experimental.pallas import tpu as pltpu

B = 8; S = 8; NT = 4; T = 128; D = 64; K = 256; V = 20000
VT = 4096
LOG2E = 1.4426950408889634


def _body(subj_ref, task_ref, xp, fmu, fsig, smu, ssig, tmu, tsig,
          epsF, epsP, epsS, wc0, wc1, wc2, ww, wtop, wbot, y_ref,
          caug_s, wts_s):
    v = pl.program_id(1)

    @pl.when(v == 0)
    def _precompute():
        z_f = fmu[0] + fsig[0] * epsF[0]
        z_p = smu[0] + ssig[0] * epsP[0]
        z_s = tmu[0] + tsig[0] * epsS[0]

        i0 = jax.lax.broadcasted_iota(jnp.int32, (D, D), 0)
        i1 = jax.lax.broadcasted_iota(jnp.int32, (D, D), 1)
        eye = jnp.where(i0 == i1, 1.0, 0.0).astype(jnp.float32)
        z_col = jnp.sum(jnp.broadcast_to(z_f, (D, D)) * eye, axis=1,
                        keepdims=True)

        f32 = jnp.float32
        c0 = jnp.dot(wc0[...], z_col, preferred_element_type=f32)
        c1 = jnp.dot(wc1[...], z_col, preferred_element_type=f32)
        c2 = jnp.dot(wc2[...], z_col, preferred_element_type=f32)
        logw = jnp.dot(ww[...], z_col, preferred_element_type=f32) + 2.0
        invw = jnp.exp(-logw)

        iwl = LOG2E * invw
        iw2 = 2.0 * iwl
        cenw = -iwl * (c0 * c0 + c1 * c1 + c2 * c2)
        zcol3 = jnp.zeros((K, 3), dtype=f32)
        caug_s[...] = jnp.concatenate(
            [iw2 * c0, iw2 * c1, iw2 * c2, -iwl, cenw, zcol3], axis=1)

        wrow = jnp.dot(z_p, wtop[...], preferred_element_type=f32)
        wts_s[...] = jnp.dot(z_s, wbot[...], preferred_element_type=f32) + wrow

    x = xp[...]
    ls = jnp.sum(x * x, axis=0, keepdims=True)
    ones = jnp.ones((1, VT), dtype=jnp.float32)
    zrow3 = jnp.zeros((3, VT), dtype=jnp.float32)
    xaug = jnp.concatenate([x[0:3], ls, ones, zrow3], axis=0)

    arg = jnp.dot(caug_s[...], xaug, preferred_element_type=jnp.float32)
    f = jnp.exp2(arg)
    y_ref[0] = jnp.dot(wts_s[...], f, preferred_element_type=jnp.float32)


def kernel(locations, block_subjects, block_tasks, factors_mu, factors_sigma,
           subject_mu, subject_sigma, task_mu, task_sigma, eps_F, eps_P,
           eps_S, W_c, W_w, W_wt):
    locT = jnp.pad(locations.T, ((0, 5), (0, 0)))
    wc = W_c.reshape(D, K, 3)
    wc0T = wc[:, :, 0].T
    wc1T = wc[:, :, 1].T
    wc2T = wc[:, :, 2].T
    wwT = W_w.T
    wtop = W_wt[:D]
    wbot = W_wt[D:]
    fmu3 = factors_mu[:, None, :]
    fsig3 = factors_sigma[:, None, :]
    smu3 = subject_mu[:, None, :]
    ssig3 = subject_sigma[:, None, :]
    epsF3 = eps_F[:, None, :]
    epsP3 = eps_P[:, None, :]

    nv = pl.cdiv(V, VT)
    grid_spec = pltpu.PrefetchScalarGridSpec(
        num_scalar_prefetch=2,
        grid=(B, nv),
        in_specs=[
            pl.BlockSpec((8, VT), lambda b, v, s, t: (0, v)),
            pl.BlockSpec((1, 1, D), lambda b, v, s, t: (s[b], 0, 0)),
            pl.BlockSpec((1, 1, D), lambda b, v, s, t: (s[b], 0, 0)),
            pl.BlockSpec((1, 1, D), lambda b, v, s, t: (s[b], 0, 0)),
            pl.BlockSpec((1, 1, D), lambda b, v, s, t: (s[b], 0, 0)),
            pl.BlockSpec((1, T, D), lambda b, v, s, t: (t[b], 0, 0)),
            pl.BlockSpec((1, T, D), lambda b, v, s, t: (t[b], 0, 0)),
            pl.BlockSpec((1, 1, D), lambda b, v, s, t: (b, 0, 0)),
            pl.BlockSpec((1, 1, D), lambda b, v, s, t: (b, 0, 0)),
            pl.BlockSpec((1, T, D), lambda b, v, s, t: (b, 0, 0)),
            pl.BlockSpec((K, D), lambda b, v, s, t: (0, 0)),
            pl.BlockSpec((K, D), lambda b, v, s, t: (0, 0)),
            pl.BlockSpec((K, D), lambda b, v, s, t: (0, 0)),
            pl.BlockSpec((K, D), lambda b, v, s, t: (0, 0)),
            pl.BlockSpec((D, K), lambda b, v, s, t: (0, 0)),
            pl.BlockSpec((D, K), lambda b, v, s, t: (0, 0)),
        ],
        out_specs=pl.BlockSpec((1, T, VT), lambda b, v, s, t: (b, 0, v)),
        scratch_shapes=[
            pltpu.VMEM((K, 8), jnp.float32),
            pltpu.VMEM((T, K), jnp.float32),
        ],
    )
    y = pl.pallas_call(
        _body,
        grid_spec=grid_spec,
        out_shape=jax.ShapeDtypeStruct((B, T, V), jnp.float32),
        compiler_params=pltpu.CompilerParams(
            dimension_semantics=("parallel", "arbitrary"),
        ),
    )(block_subjects, block_tasks, locT, fmu3, fsig3, smu3, ssig3,
      task_mu, task_sigma, epsF3, epsP3, eps_S, wc0T, wc1T, wc2T, wwT,
      wtop, wbot)
    return y


# speedup vs baseline: 1.5866x; 1.0318x over previous
import jax
import jax.numpy as jnp
from jax.experimental import pallas as pl
from jax.experimental.pallas import tpu as pltpu

B = 8; S = 8; NT = 4; T = 128; D = 64; K = 256; V = 20000
VT = 5120
LOG2E = 1.4426950408889634


def _body(subj_ref, task_ref, xp, fmu, fsig, smu, ssig, tmu, tsig,
          epsF, epsP, epsS, wc0, wc1, wc2, ww, wtop, wbot, y_ref,
          caug_s, wts_s):
    v = pl.program_id(1)

    @pl.when(v == 0)
    def _precompute():
        z_f = fmu[0] + fsig[0] * epsF[0]
        z_p = smu[0] + ssig[0] * epsP[0]
        z_s = tmu[0] + tsig[0] * epsS[0]

        i0 = jax.lax.broadcasted_iota(jnp.int32, (D, D), 0)
        i1 = jax.lax.broadcasted_iota(jnp.int32, (D, D), 1)
        eye = jnp.where(i0 == i1, 1.0, 0.0).astype(jnp.float32)
        z_col = jnp.sum(jnp.broadcast_to(z_f, (D, D)) * eye, axis=1,
                        keepdims=True)

        f32 = jnp.float32
        c0 = jnp.dot(wc0[...], z_col, preferred_element_type=f32)
        c1 = jnp.dot(wc1[...], z_col, preferred_element_type=f32)
        c2 = jnp.dot(wc2[...], z_col, preferred_element_type=f32)
        logw = jnp.dot(ww[...], z_col, preferred_element_type=f32) + 2.0
        invw = jnp.exp(-logw)

        iwl = LOG2E * invw
        iw2 = 2.0 * iwl
        cenw = -iwl * (c0 * c0 + c1 * c1 + c2 * c2)
        zcol3 = jnp.zeros((K, 3), dtype=f32)
        caug_s[...] = jnp.concatenate(
            [iw2 * c0, iw2 * c1, iw2 * c2, -iwl, cenw, zcol3], axis=1)

        wrow = jnp.dot(z_p, wtop[...], preferred_element_type=f32)
        wts_s[...] = jnp.dot(z_s, wbot[...], preferred_element_type=f32) + wrow

    x = xp[...]
    ls = jnp.sum(x * x, axis=0, keepdims=True)
    ones = jnp.ones((1, VT), dtype=jnp.float32)
    zrow3 = jnp.zeros((3, VT), dtype=jnp.float32)
    xaug = jnp.concatenate([x[0:3], ls, ones, zrow3], axis=0)

    arg = jnp.dot(caug_s[...], xaug, preferred_element_type=jnp.float32)
    f = jnp.exp2(arg)
    y_ref[0] = jnp.dot(wts_s[...], f, preferred_element_type=jnp.float32)


def kernel(locations, block_subjects, block_tasks, factors_mu, factors_sigma,
           subject_mu, subject_sigma, task_mu, task_sigma, eps_F, eps_P,
           eps_S, W_c, W_w, W_wt):
    locT = jnp.pad(locations.T, ((0, 5), (0, 0)))
    wc = W_c.reshape(D, K, 3)
    wc0T = wc[:, :, 0].T
    wc1T = wc[:, :, 1].T
    wc2T = wc[:, :, 2].T
    wwT = W_w.T
    wtop = W_wt[:D]
    wbot = W_wt[D:]
    fmu3 = factors_mu[:, None, :]
    fsig3 = factors_sigma[:, None, :]
    smu3 = subject_mu[:, None, :]
    ssig3 = subject_sigma[:, None, :]
    epsF3 = eps_F[:, None, :]
    epsP3 = eps_P[:, None, :]

    nv = pl.cdiv(V, VT)
    grid_spec = pltpu.PrefetchScalarGridSpec(
        num_scalar_prefetch=2,
        grid=(B, nv),
        in_specs=[
            pl.BlockSpec((8, VT), lambda b, v, s, t: (0, v)),
            pl.BlockSpec((1, 1, D), lambda b, v, s, t: (s[b], 0, 0)),
            pl.BlockSpec((1, 1, D), lambda b, v, s, t: (s[b], 0, 0)),
            pl.BlockSpec((1, 1, D), lambda b, v, s, t: (s[b], 0, 0)),
            pl.BlockSpec((1, 1, D), lambda b, v, s, t: (s[b], 0, 0)),
            pl.BlockSpec((1, T, D), lambda b, v, s, t: (t[b], 0, 0)),
            pl.BlockSpec((1, T, D), lambda b, v, s, t: (t[b], 0, 0)),
            pl.BlockSpec((1, 1, D), lambda b, v, s, t: (b, 0, 0)),
            pl.BlockSpec((1, 1, D), lambda b, v, s, t: (b, 0, 0)),
            pl.BlockSpec((1, T, D), lambda b, v, s, t: (b, 0, 0)),
            pl.BlockSpec((K, D), lambda b, v, s, t: (0, 0)),
            pl.BlockSpec((K, D), lambda b, v, s, t: (0, 0)),
            pl.BlockSpec((K, D), lambda b, v, s, t: (0, 0)),
            pl.BlockSpec((K, D), lambda b, v, s, t: (0, 0)),
            pl.BlockSpec((D, K), lambda b, v, s, t: (0, 0)),
            pl.BlockSpec((D, K), lambda b, v, s, t: (0, 0)),
        ],
        out_specs=pl.BlockSpec((1, T, VT), lambda b, v, s, t: (b, 0, v)),
        scratch_shapes=[
            pltpu.VMEM((K, 8), jnp.float32),
            pltpu.VMEM((T, K), jnp.float32),
        ],
    )
    y = pl.pallas_call(
        _body,
        grid_spec=grid_spec,
        out_shape=jax.ShapeDtypeStruct((B, T, V), jnp.float32),
        compiler_params=pltpu.CompilerParams(
            dimension_semantics=("parallel", "arbitrary"),
        ),
    )(block_subjects, block_tasks, locT, fmu3, fsig3, smu3, ssig3,
      task_mu, task_sigma, epsF3, epsP3, eps_S, wc0T, wc1T, wc2T, wwT,
      wtop, wbot)
    return y


# speedup vs baseline: 1.6735x; 1.0547x over previous
import jax
import jax.numpy as jnp
from jax.experimental import pallas as pl
from jax.experimental.pallas import tpu as pltpu

B = 8; S = 8; NT = 4; T = 128; D = 64; K = 256; V = 20000
VT = 10112
LOG2E = 1.4426950408889634


def _body(subj_ref, task_ref, xp, fmu, fsig, smu, ssig, tmu, tsig,
          epsF, epsP, epsS, wc0, wc1, wc2, ww, wtop, wbot, y_ref,
          caug_s, wts_s):
    v = pl.program_id(1)

    @pl.when(v == 0)
    def _precompute():
        z_f = fmu[0] + fsig[0] * epsF[0]
        z_p = smu[0] + ssig[0] * epsP[0]
        z_s = tmu[0] + tsig[0] * epsS[0]

        i0 = jax.lax.broadcasted_iota(jnp.int32, (D, D), 0)
        i1 = jax.lax.broadcasted_iota(jnp.int32, (D, D), 1)
        eye = jnp.where(i0 == i1, 1.0, 0.0).astype(jnp.float32)
        z_col = jnp.sum(jnp.broadcast_to(z_f, (D, D)) * eye, axis=1,
                        keepdims=True)

        f32 = jnp.float32
        c0 = jnp.dot(wc0[...], z_col, preferred_element_type=f32)
        c1 = jnp.dot(wc1[...], z_col, preferred_element_type=f32)
        c2 = jnp.dot(wc2[...], z_col, preferred_element_type=f32)
        logw = jnp.dot(ww[...], z_col, preferred_element_type=f32) + 2.0
        invw = jnp.exp(-logw)

        iwl = LOG2E * invw
        iw2 = 2.0 * iwl
        cenw = -iwl * (c0 * c0 + c1 * c1 + c2 * c2)
        zcol3 = jnp.zeros((K, 3), dtype=f32)
        caug_s[...] = jnp.concatenate(
            [iw2 * c0, iw2 * c1, iw2 * c2, -iwl, cenw, zcol3], axis=1)

        wrow = jnp.dot(z_p, wtop[...], preferred_element_type=f32)
        wts_s[...] = jnp.dot(z_s, wbot[...], preferred_element_type=f32) + wrow

    x = xp[...]
    ls = jnp.sum(x * x, axis=0, keepdims=True)
    ones = jnp.ones((1, VT), dtype=jnp.float32)
    zrow3 = jnp.zeros((3, VT), dtype=jnp.float32)
    xaug = jnp.concatenate([x[0:3], ls, ones, zrow3], axis=0)

    arg = jnp.dot(caug_s[...], xaug, preferred_element_type=jnp.float32)
    f = jnp.exp2(arg)
    y_ref[0] = jnp.dot(wts_s[...], f, preferred_element_type=jnp.float32)


def kernel(locations, block_subjects, block_tasks, factors_mu, factors_sigma,
           subject_mu, subject_sigma, task_mu, task_sigma, eps_F, eps_P,
           eps_S, W_c, W_w, W_wt):
    locT = jnp.pad(locations.T, ((0, 5), (0, 0)))
    wc = W_c.reshape(D, K, 3)
    wc0T = wc[:, :, 0].T
    wc1T = wc[:, :, 1].T
    wc2T = wc[:, :, 2].T
    wwT = W_w.T
    wtop = W_wt[:D]
    wbot = W_wt[D:]
    fmu3 = factors_mu[:, None, :]
    fsig3 = factors_sigma[:, None, :]
    smu3 = subject_mu[:, None, :]
    ssig3 = subject_sigma[:, None, :]
    epsF3 = eps_F[:, None, :]
    epsP3 = eps_P[:, None, :]

    nv = pl.cdiv(V, VT)
    grid_spec = pltpu.PrefetchScalarGridSpec(
        num_scalar_prefetch=2,
        grid=(B, nv),
        in_specs=[
            pl.BlockSpec((8, VT), lambda b, v, s, t: (0, v)),
            pl.BlockSpec((1, 1, D), lambda b, v, s, t: (s[b], 0, 0)),
            pl.BlockSpec((1, 1, D), lambda b, v, s, t: (s[b], 0, 0)),
            pl.BlockSpec((1, 1, D), lambda b, v, s, t: (s[b], 0, 0)),
            pl.BlockSpec((1, 1, D), lambda b, v, s, t: (s[b], 0, 0)),
            pl.BlockSpec((1, T, D), lambda b, v, s, t: (t[b], 0, 0)),
            pl.BlockSpec((1, T, D), lambda b, v, s, t: (t[b], 0, 0)),
            pl.BlockSpec((1, 1, D), lambda b, v, s, t: (b, 0, 0)),
            pl.BlockSpec((1, 1, D), lambda b, v, s, t: (b, 0, 0)),
            pl.BlockSpec((1, T, D), lambda b, v, s, t: (b, 0, 0)),
            pl.BlockSpec((K, D), lambda b, v, s, t: (0, 0)),
            pl.BlockSpec((K, D), lambda b, v, s, t: (0, 0)),
            pl.BlockSpec((K, D), lambda b, v, s, t: (0, 0)),
            pl.BlockSpec((K, D), lambda b, v, s, t: (0, 0)),
            pl.BlockSpec((D, K), lambda b, v, s, t: (0, 0)),
            pl.BlockSpec((D, K), lambda b, v, s, t: (0, 0)),
        ],
        out_specs=pl.BlockSpec((1, T, VT), lambda b, v, s, t: (b, 0, v)),
        scratch_shapes=[
            pltpu.VMEM((K, 8), jnp.float32),
            pltpu.VMEM((T, K), jnp.float32),
        ],
    )
    y = pl.pallas_call(
        _body,
        grid_spec=grid_spec,
        out_shape=jax.ShapeDtypeStruct((B, T, V), jnp.float32),
        compiler_params=pltpu.CompilerParams(
            dimension_semantics=("parallel", "arbitrary"),
        ),
    )(block_subjects, block_tasks, locT, fmu3, fsig3, smu3, ssig3,
      task_mu, task_sigma, epsF3, epsP3, eps_S, wc0T, wc1T, wc2T, wwT,
      wtop, wbot)
    return y
